# baseline (speedup 1.0000x reference)
import functools

import jax
import jax.numpy as jnp
import numpy as np
from jax import lax
from jax.experimental import pallas as pl
from jax.experimental.pallas import tpu as pltpu
from jax.experimental.pallas import tpu_sc as plsc

B = 1
S = 2048
HID = 2048
QLORA = 1536
H = 32
D = 128
ROPE = 64
HALF = ROPE // 2
TOPK = 1024

QB = 512

NC = 2
NS = 16
NW = NC * NS
RPW = S // NW
LANES = 16
NCHUNK = S // LANES
NBINS = 64
RADIX_SHIFTS = (0, 6, 12, 18, 24, 30)

_C = float(D) ** -0.5


def _hadamard(x, scale):
    orig_dtype = x.dtype
    x = x.astype(jnp.float32)
    dim = x.shape[-1]
    h = 1
    while h < dim:
        x = x.reshape(x.shape[:-1] + (dim // (2 * h), 2, h))
        a = x[..., 0, :]
        b = x[..., 1, :]
        x = jnp.stack([a + b, a - b], axis=-2)
        x = x.reshape(x.shape[:-3] + (dim,))
        h *= 2
    return (x * scale).astype(orig_dtype)


def _rotated_qk(hidden_states, q_compressed, cos, sin, Wq, Wk, ln_g, ln_b, Ww):
    b, s, _ = hidden_states.shape
    q = q_compressed @ Wq
    q = q.reshape(b, s, H, D)
    q_rope, q_nope = q[..., :ROPE], q[..., ROPE:]
    k = hidden_states @ Wk
    mu = jnp.mean(k, axis=-1, keepdims=True)
    var = jnp.mean((k - mu) ** 2, axis=-1, keepdims=True)
    k = (k - mu) / jnp.sqrt(var + 1e-5) * ln_g + ln_b
    k_rope, k_nope = k[..., :ROPE], k[..., ROPE:]
    k_rope = k_rope[:, :, None, :]
    cosu = cos[:, None, :]
    sinu = sin[:, None, :]
    q1, q2 = jnp.split(q_rope, 2, axis=-1)
    k1, k2 = jnp.split(k_rope, 2, axis=-1)
    q_rope = jnp.concatenate([q1 * cosu - q2 * sinu, q1 * sinu + q2 * cosu],
                             axis=-1)
    k_rope = jnp.concatenate([k1 * cosu - k2 * sinu, k1 * sinu + k2 * cosu],
                             axis=-1)
    k_rope = k_rope[:, :, 0, :]
    q = jnp.concatenate([q_rope, q_nope], axis=-1)
    k = jnp.concatenate([k_rope, k_nope], axis=-1)
    q = _hadamard(q, _C)
    k = _hadamard(k, _C)
    w = (hidden_states.astype(jnp.float32) @ Ww) * (H ** -0.5)
    qb = q.astype(jnp.bfloat16).reshape(s, H * D)
    kb = k.astype(jnp.bfloat16).reshape(s, D)
    return qb, kb, w.reshape(s, H)


def _scores_body(qb_ref, kb_ref, w_ref, out_ref):
    kb = kb_ref[...]
    w = w_ref[...]
    acc = None
    for h in range(H):
        qh = qb_ref[:, h * D:(h + 1) * D]
        sc = lax.dot_general(qh, kb, (((1,), (1,)), ((), ())),
                             preferred_element_type=jnp.float32)
        term = (jnp.maximum(sc, 0.0) * _C) * w[:, h:h + 1]
        acc = term if acc is None else acc + term
    bits = lax.bitcast_convert_type(acc, jnp.int32)
    bits = jnp.where(acc == 0.0, 0, bits)
    out_ref[...] = jnp.where(acc >= 0.0, jnp.int32(0x7FFFFFFF) - bits, bits)


def _index_scores_keys(hidden_states, q_compressed, cos, sin, Wq, Wk,
                       ln_g, ln_b, Ww):
    qb, kb, w = _rotated_qk(hidden_states, q_compressed, cos, sin, Wq, Wk,
                            ln_g, ln_b, Ww)
    keys = pl.pallas_call(
        _scores_body,
        grid=(S // QB,),
        in_specs=[
            pl.BlockSpec((QB, H * D), lambda i: (i, 0)),
            pl.BlockSpec((S, D), lambda i: (0, 0)),
            pl.BlockSpec((QB, H), lambda i: (i, 0)),
        ],
        out_specs=pl.BlockSpec((QB, S), lambda i: (i, 0)),
        out_shape=jax.ShapeDtypeStruct((S, S), jnp.int32),
    )(qb, kb, w)
    return keys


def _sc_topk_body(keys_hbm, out_hbm, ka0, va0, kb0, vb0, c0, ka1, va1, kb1,
                  vb1, c1, sem):
    wid = lax.axis_index("s") * NC + lax.axis_index("c")
    lane = lax.iota(jnp.int32, LANES)
    zeros16 = jnp.zeros((LANES,), jnp.int32)
    ones16 = jnp.ones((LANES,), jnp.int32)
    cnts = (c0, c1)

    def radix_pass(srcs, dsts, shift, last):
        shift_v = jnp.full((LANES,), shift, jnp.int32)

        @pl.loop(0, NBINS, unroll=8)
        def _(j):
            for cnt in cnts:
                cnt[pl.ds(j * LANES, LANES)] = zeros16

        @pl.loop(0, NCHUNK, unroll=4)
        def _(ci):
            for (src_k, _), cnt in zip(srcs, cnts):
                k = src_k[pl.ds(ci * LANES, LANES)]
                d = lax.shift_right_logical(k, shift_v) & 63
                plsc.addupdate_scatter(cnt, [d * LANES + lane], ones16)

        def scan_body(j, carry):
            nxt = []
            for cnt, car in zip(cnts, carry):
                v = cnt[pl.ds(j * LANES, LANES)]
                ex = plsc.cumsum(v) - v + car
                cnt[pl.ds(j * LANES, LANES)] = ex
                nxt.append(car + jnp.sum(v))
            return tuple(nxt)

        lax.fori_loop(0, NBINS, scan_body, (jnp.int32(0), jnp.int32(0)),
                      unroll=4)

        @pl.loop(0, NCHUNK, unroll=4)
        def _(ci):
            for (src_k, src_v), (dst_k, dst_v), cnt in zip(srcs, dsts, cnts):
                k = src_k[pl.ds(ci * LANES, LANES)]
                v = src_v[pl.ds(ci * LANES, LANES)]
                d = lax.shift_right_logical(k, shift_v) & 63
                addr = d * LANES + lane
                slot = plsc.load_gather(cnt, [addr])
                plsc.store_scatter(cnt, [addr], slot + 1)
                if last:
                    pos = slot
                else:
                    pos = (slot & (NCHUNK - 1)) * LANES + (slot >> 7)
                plsc.store_scatter(dst_k, [pos], k)
                plsc.store_scatter(dst_v, [pos], v)

    @pl.loop(0, RPW // 2)
    def _(ri):
        r = wid * RPW + ri * 2
        pltpu.sync_copy(keys_hbm.at[r], ka0)
        pltpu.sync_copy(keys_hbm.at[r + 1], ka1)

        @pl.loop(0, NCHUNK, unroll=8)
        def _(ci):
            va0[pl.ds(ci * LANES, LANES)] = lane + ci * LANES
            va1[pl.ds(ci * LANES, LANES)] = lane + ci * LANES

        a_set = ((ka0, va0), (ka1, va1))
        b_set = ((kb0, vb0), (kb1, vb1))
        np_ = len(RADIX_SHIFTS)
        for p in range(0, np_, 2):
            radix_pass(a_set, b_set, RADIX_SHIFTS[p], False)
            radix_pass(b_set, a_set, RADIX_SHIFTS[p + 1], p + 2 == np_)

        pltpu.sync_copy(va0.at[pl.ds(0, TOPK)], out_hbm.at[r])
        pltpu.sync_copy(va1.at[pl.ds(0, TOPK)], out_hbm.at[r + 1])


def _sc_topk(keys):
    mesh = plsc.VectorSubcoreMesh(core_axis_name="c", subcore_axis_name="s",
                                  num_cores=NC, num_subcores=NS)
    f = pl.kernel(
        _sc_topk_body,
        out_type=jax.ShapeDtypeStruct((S, TOPK), jnp.int32),
        mesh=mesh,
        compiler_params=pltpu.CompilerParams(needs_layout_passes=False),
        scratch_types=[
            pltpu.VMEM((S,), jnp.int32),
            pltpu.VMEM((S,), jnp.int32),
            pltpu.VMEM((S,), jnp.int32),
            pltpu.VMEM((S,), jnp.int32),
            pltpu.VMEM((NBINS * LANES,), jnp.int32),
            pltpu.VMEM((S,), jnp.int32),
            pltpu.VMEM((S,), jnp.int32),
            pltpu.VMEM((S,), jnp.int32),
            pltpu.VMEM((S,), jnp.int32),
            pltpu.VMEM((NBINS * LANES,), jnp.int32),
            pltpu.SemaphoreType.DMA,
        ],
    )
    return f(keys)


def kernel(hidden_states, q_compressed, cos, sin, Wq, Wk, ln_g, ln_b, Ww):
    keys = _index_scores_keys(hidden_states, q_compressed, cos, sin, Wq, Wk,
                              ln_g, ln_b, Ww)
    idx = _sc_topk(keys)
    return idx.reshape(B, S, TOPK)


# speedup vs baseline: 1.0541x; 1.0541x over previous
import functools

import jax
import jax.numpy as jnp
import numpy as np
from jax import lax
from jax.experimental import pallas as pl
from jax.experimental.pallas import tpu as pltpu
from jax.experimental.pallas import tpu_sc as plsc

B = 1
S = 2048
HID = 2048
QLORA = 1536
H = 32
D = 128
ROPE = 64
HALF = ROPE // 2
TOPK = 1024

QB = 512

NC = 2
NS = 16
NW = NC * NS
RPW = S // NW
LANES = 16
NCHUNK = S // LANES
NBINS = 64
RADIX_SHIFTS = (0, 6, 12, 18, 24, 30)

_C = float(D) ** -0.5


def _hadamard(x, scale):
    orig_dtype = x.dtype
    x = x.astype(jnp.float32)
    dim = x.shape[-1]
    h = 1
    while h < dim:
        x = x.reshape(x.shape[:-1] + (dim // (2 * h), 2, h))
        a = x[..., 0, :]
        b = x[..., 1, :]
        x = jnp.stack([a + b, a - b], axis=-2)
        x = x.reshape(x.shape[:-3] + (dim,))
        h *= 2
    return (x * scale).astype(orig_dtype)


def _rotated_qk(hidden_states, q_compressed, cos, sin, Wq, Wk, ln_g, ln_b, Ww):
    b, s, _ = hidden_states.shape
    q = q_compressed @ Wq
    q = q.reshape(b, s, H, D)
    q_rope, q_nope = q[..., :ROPE], q[..., ROPE:]
    k = hidden_states @ Wk
    mu = jnp.mean(k, axis=-1, keepdims=True)
    var = jnp.mean((k - mu) ** 2, axis=-1, keepdims=True)
    k = (k - mu) / jnp.sqrt(var + 1e-5) * ln_g + ln_b
    k_rope, k_nope = k[..., :ROPE], k[..., ROPE:]
    k_rope = k_rope[:, :, None, :]
    cosu = cos[:, None, :]
    sinu = sin[:, None, :]
    q1, q2 = jnp.split(q_rope, 2, axis=-1)
    k1, k2 = jnp.split(k_rope, 2, axis=-1)
    q_rope = jnp.concatenate([q1 * cosu - q2 * sinu, q1 * sinu + q2 * cosu],
                             axis=-1)
    k_rope = jnp.concatenate([k1 * cosu - k2 * sinu, k1 * sinu + k2 * cosu],
                             axis=-1)
    k_rope = k_rope[:, :, 0, :]
    q = jnp.concatenate([q_rope, q_nope], axis=-1)
    k = jnp.concatenate([k_rope, k_nope], axis=-1)
    q = _hadamard(q, _C)
    k = _hadamard(k, _C)
    w = (hidden_states.astype(jnp.float32) @ Ww) * (H ** -0.5)
    qb = q.astype(jnp.bfloat16).reshape(s, H * D)
    kb = k.astype(jnp.bfloat16).reshape(s, D)
    return qb, kb, w.reshape(s, H)


def _scores_body(qb_ref, kb_ref, w_ref, out_ref):
    kb = kb_ref[...]
    w = w_ref[...]
    acc = None
    for h in range(H):
        qh = qb_ref[:, h * D:(h + 1) * D]
        sc = lax.dot_general(qh, kb, (((1,), (1,)), ((), ())),
                             preferred_element_type=jnp.float32)
        term = (jnp.maximum(sc, 0.0) * _C) * w[:, h:h + 1]
        acc = term if acc is None else acc + term
    bits = lax.bitcast_convert_type(acc, jnp.int32)
    bits = jnp.where(acc == 0.0, 0, bits)
    out_ref[...] = jnp.where(acc >= 0.0, jnp.int32(0x7FFFFFFF) - bits, bits)


def _index_scores_keys(hidden_states, q_compressed, cos, sin, Wq, Wk,
                       ln_g, ln_b, Ww):
    qb, kb, w = _rotated_qk(hidden_states, q_compressed, cos, sin, Wq, Wk,
                            ln_g, ln_b, Ww)
    keys = pl.pallas_call(
        _scores_body,
        grid=(S // QB,),
        in_specs=[
            pl.BlockSpec((QB, H * D), lambda i: (i, 0)),
            pl.BlockSpec((S, D), lambda i: (0, 0)),
            pl.BlockSpec((QB, H), lambda i: (i, 0)),
        ],
        out_specs=pl.BlockSpec((QB, S), lambda i: (i, 0)),
        out_shape=jax.ShapeDtypeStruct((S, S), jnp.int32),
    )(qb, kb, w)
    return keys


def _sc_topk_body(keys_hbm, out_hbm, key_a, val_a, key_b, val_b, cnt, tot,
                  sem):
    wid = lax.axis_index("s") * NC + lax.axis_index("c")
    lane = lax.iota(jnp.int32, LANES)
    zeros16 = jnp.zeros((LANES,), jnp.int32)
    ones16 = jnp.ones((LANES,), jnp.int32)

    def radix_pass(src_k, src_v, dst_k, dst_v, shift, last):
        shift_v = jnp.full((LANES,), shift, jnp.int32)

        @pl.loop(0, NBINS, unroll=8)
        def _(j):
            cnt[pl.ds(j * LANES, LANES)] = zeros16

        @pl.loop(0, NCHUNK, unroll=8)
        def _(ci):
            k = src_k[pl.ds(ci * LANES, LANES)]
            d = lax.shift_right_logical(k, shift_v) & 63
            plsc.addupdate_scatter(cnt, [d * LANES + lane], ones16)

        @pl.loop(0, NBINS, unroll=8)
        def _(j):
            tot[j] = jnp.sum(cnt[pl.ds(j * LANES, LANES)])

        def tscan(j, carry):
            t = tot[j]
            tot[j] = carry
            return carry + t

        lax.fori_loop(0, NBINS, tscan, jnp.int32(0), unroll=8)

        @pl.loop(0, NBINS, unroll=8)
        def _(j):
            v = cnt[pl.ds(j * LANES, LANES)]
            cnt[pl.ds(j * LANES, LANES)] = plsc.cumsum(v) - v + tot[j]

        @pl.loop(0, NCHUNK, unroll=8)
        def _(ci):
            k = src_k[pl.ds(ci * LANES, LANES)]
            v = src_v[pl.ds(ci * LANES, LANES)]
            d = lax.shift_right_logical(k, shift_v) & 63
            addr = d * LANES + lane
            slot = plsc.load_gather(cnt, [addr])
            plsc.store_scatter(cnt, [addr], slot + 1)
            if last:
                pos = slot
            else:
                pos = (slot & (NCHUNK - 1)) * LANES + (slot >> 7)
            plsc.store_scatter(dst_k, [pos], k)
            plsc.store_scatter(dst_v, [pos], v)

    @pl.loop(0, RPW)
    def _(ri):
        r = wid * RPW + ri
        pltpu.sync_copy(keys_hbm.at[r], key_a)

        @pl.loop(0, NCHUNK, unroll=8)
        def _(ci):
            val_a[pl.ds(ci * LANES, LANES)] = lane + ci * LANES

        np_ = len(RADIX_SHIFTS)
        for p in range(0, np_, 2):
            radix_pass(key_a, val_a, key_b, val_b, RADIX_SHIFTS[p], False)
            radix_pass(key_b, val_b, key_a, val_a, RADIX_SHIFTS[p + 1],
                       p + 2 == np_)

        pltpu.sync_copy(val_a.at[pl.ds(0, TOPK)], out_hbm.at[r])


def _sc_topk(keys):
    mesh = plsc.VectorSubcoreMesh(core_axis_name="c", subcore_axis_name="s",
                                  num_cores=NC, num_subcores=NS)
    f = pl.kernel(
        _sc_topk_body,
        out_type=jax.ShapeDtypeStruct((S, TOPK), jnp.int32),
        mesh=mesh,
        compiler_params=pltpu.CompilerParams(needs_layout_passes=False),
        scratch_types=[
            pltpu.VMEM((S,), jnp.int32),
            pltpu.VMEM((S,), jnp.int32),
            pltpu.VMEM((S,), jnp.int32),
            pltpu.VMEM((S,), jnp.int32),
            pltpu.VMEM((NBINS * LANES,), jnp.int32),
            pltpu.SMEM((NBINS,), jnp.int32),
            pltpu.SemaphoreType.DMA,
        ],
    )
    return f(keys)


def kernel(hidden_states, q_compressed, cos, sin, Wq, Wk, ln_g, ln_b, Ww):
    keys = _index_scores_keys(hidden_states, q_compressed, cos, sin, Wq, Wk,
                              ln_g, ln_b, Ww)
    idx = _sc_topk(keys)
    return idx.reshape(B, S, TOPK)


# speedup vs baseline: 1.0698x; 1.0149x over previous
import functools

import jax
import jax.numpy as jnp
import numpy as np
from jax import lax
from jax.experimental import pallas as pl
from jax.experimental.pallas import tpu as pltpu
from jax.experimental.pallas import tpu_sc as plsc

B = 1
S = 2048
HID = 2048
QLORA = 1536
H = 32
D = 128
ROPE = 64
HALF = ROPE // 2
TOPK = 1024

QB = 512

NC = 2
NS = 16
NW = NC * NS
RPW = S // NW
LANES = 16
NCHUNK = S // LANES
NBINS = 64
RADIX_SHIFTS = (0, 6, 12, 18, 24, 30)

_C = float(D) ** -0.5


def _hadamard(x, scale):
    orig_dtype = x.dtype
    x = x.astype(jnp.float32)
    dim = x.shape[-1]
    h = 1
    while h < dim:
        x = x.reshape(x.shape[:-1] + (dim // (2 * h), 2, h))
        a = x[..., 0, :]
        b = x[..., 1, :]
        x = jnp.stack([a + b, a - b], axis=-2)
        x = x.reshape(x.shape[:-3] + (dim,))
        h *= 2
    return (x * scale).astype(orig_dtype)


def _rotated_qk(hidden_states, q_compressed, cos, sin, Wq, Wk, ln_g, ln_b, Ww):
    b, s, _ = hidden_states.shape
    q = q_compressed @ Wq
    q = q.reshape(b, s, H, D)
    q_rope, q_nope = q[..., :ROPE], q[..., ROPE:]
    k = hidden_states @ Wk
    mu = jnp.mean(k, axis=-1, keepdims=True)
    var = jnp.mean((k - mu) ** 2, axis=-1, keepdims=True)
    k = (k - mu) / jnp.sqrt(var + 1e-5) * ln_g + ln_b
    k_rope, k_nope = k[..., :ROPE], k[..., ROPE:]
    k_rope = k_rope[:, :, None, :]
    cosu = cos[:, None, :]
    sinu = sin[:, None, :]
    q1, q2 = jnp.split(q_rope, 2, axis=-1)
    k1, k2 = jnp.split(k_rope, 2, axis=-1)
    q_rope = jnp.concatenate([q1 * cosu - q2 * sinu, q1 * sinu + q2 * cosu],
                             axis=-1)
    k_rope = jnp.concatenate([k1 * cosu - k2 * sinu, k1 * sinu + k2 * cosu],
                             axis=-1)
    k_rope = k_rope[:, :, 0, :]
    q = jnp.concatenate([q_rope, q_nope], axis=-1)
    k = jnp.concatenate([k_rope, k_nope], axis=-1)
    q = _hadamard(q, _C)
    k = _hadamard(k, _C)
    w = (hidden_states.astype(jnp.float32) @ Ww) * (H ** -0.5)
    qb = q.astype(jnp.bfloat16).reshape(s, H * D)
    kb = k.astype(jnp.bfloat16).reshape(s, D)
    return qb, kb, w.reshape(s, H)


def _scores_body(qb_ref, kb_ref, w_ref, out_ref):
    kb = kb_ref[...]
    w = w_ref[...]
    acc = None
    for h in range(H):
        qh = qb_ref[:, h * D:(h + 1) * D]
        sc = lax.dot_general(qh, kb, (((1,), (1,)), ((), ())),
                             preferred_element_type=jnp.float32)
        term = (jnp.maximum(sc, 0.0) * _C) * w[:, h:h + 1]
        acc = term if acc is None else acc + term
    bits = lax.bitcast_convert_type(acc, jnp.int32)
    bits = jnp.where(acc == 0.0, 0, bits)
    out_ref[...] = jnp.where(acc >= 0.0, jnp.int32(0x7FFFFFFF) - bits, bits)


def _keys_part(qb_part, kb, w_part, nrows):
    return pl.pallas_call(
        _scores_body,
        grid=(nrows // QB,),
        in_specs=[
            pl.BlockSpec((QB, H * D), lambda i: (i, 0)),
            pl.BlockSpec((S, D), lambda i: (0, 0)),
            pl.BlockSpec((QB, H), lambda i: (i, 0)),
        ],
        out_specs=pl.BlockSpec((QB, S), lambda i: (i, 0)),
        out_shape=jax.ShapeDtypeStruct((nrows, S), jnp.int32),
    )(qb_part, kb, w_part)


def _sc_topk_body(keys_hbm, out_hbm, key_a, val_a, key_b, val_b, cnt, tot,
                  sem):
    wid = lax.axis_index("s") * NC + lax.axis_index("c")
    lane = lax.iota(jnp.int32, LANES)
    zeros16 = jnp.zeros((LANES,), jnp.int32)
    ones16 = jnp.ones((LANES,), jnp.int32)

    def radix_pass(src_k, src_v, dst_k, dst_v, shift, last):
        shift_v = jnp.full((LANES,), shift, jnp.int32)

        @pl.loop(0, NBINS, unroll=8)
        def _(j):
            cnt[pl.ds(j * LANES, LANES)] = zeros16

        @pl.loop(0, NCHUNK, unroll=8)
        def _(ci):
            k = src_k[pl.ds(ci * LANES, LANES)]
            d = lax.shift_right_logical(k, shift_v) & 63
            plsc.addupdate_scatter(cnt, [d * LANES + lane], ones16)

        @pl.loop(0, NBINS, unroll=8)
        def _(j):
            tot[j] = jnp.sum(cnt[pl.ds(j * LANES, LANES)])

        def tscan(j, carry):
            t = tot[j]
            tot[j] = carry
            return carry + t

        lax.fori_loop(0, NBINS, tscan, jnp.int32(0), unroll=8)

        @pl.loop(0, NBINS, unroll=8)
        def _(j):
            v = cnt[pl.ds(j * LANES, LANES)]
            cnt[pl.ds(j * LANES, LANES)] = plsc.cumsum(v) - v + tot[j]

        @pl.loop(0, NCHUNK, unroll=8)
        def _(ci):
            k = src_k[pl.ds(ci * LANES, LANES)]
            v = src_v[pl.ds(ci * LANES, LANES)]
            d = lax.shift_right_logical(k, shift_v) & 63
            addr = d * LANES + lane
            slot = plsc.load_gather(cnt, [addr])
            plsc.store_scatter(cnt, [addr], slot + 1)
            if last:
                pos = slot
            else:
                pos = (slot & (NCHUNK - 1)) * LANES + (slot >> 7)
            plsc.store_scatter(dst_k, [pos], k)
            plsc.store_scatter(dst_v, [pos], v)

    rpw = keys_hbm.shape[0] // NW

    @pl.loop(0, rpw)
    def _(ri):
        r = wid * rpw + ri
        pltpu.sync_copy(keys_hbm.at[r], key_a)

        @pl.loop(0, NCHUNK, unroll=8)
        def _(ci):
            val_a[pl.ds(ci * LANES, LANES)] = lane + ci * LANES

        np_ = len(RADIX_SHIFTS)
        for p in range(0, np_, 2):
            radix_pass(key_a, val_a, key_b, val_b, RADIX_SHIFTS[p], False)
            radix_pass(key_b, val_b, key_a, val_a, RADIX_SHIFTS[p + 1],
                       p + 2 == np_)

        pltpu.sync_copy(val_a.at[pl.ds(0, TOPK)], out_hbm.at[r])


def _sc_topk(keys):
    mesh = plsc.VectorSubcoreMesh(core_axis_name="c", subcore_axis_name="s",
                                  num_cores=NC, num_subcores=NS)
    f = pl.kernel(
        _sc_topk_body,
        out_type=jax.ShapeDtypeStruct((keys.shape[0], TOPK), jnp.int32),
        mesh=mesh,
        compiler_params=pltpu.CompilerParams(needs_layout_passes=False),
        scratch_types=[
            pltpu.VMEM((S,), jnp.int32),
            pltpu.VMEM((S,), jnp.int32),
            pltpu.VMEM((S,), jnp.int32),
            pltpu.VMEM((S,), jnp.int32),
            pltpu.VMEM((NBINS * LANES,), jnp.int32),
            pltpu.SMEM((NBINS,), jnp.int32),
            pltpu.SemaphoreType.DMA,
        ],
    )
    return f(keys)


def kernel(hidden_states, q_compressed, cos, sin, Wq, Wk, ln_g, ln_b, Ww):
    qb, kb, w = _rotated_qk(hidden_states, q_compressed, cos, sin, Wq, Wk,
                            ln_g, ln_b, Ww)
    half = S // 2
    parts = []
    for p in range(2):
        sl = slice(p * half, (p + 1) * half)
        keys_p = _keys_part(qb[sl], kb, w[sl], half)
        parts.append(_sc_topk(keys_p))
    idx = jnp.concatenate(parts, axis=0)
    return idx.reshape(B, S, TOPK)


# speedup vs baseline: 1.3051x; 1.2199x over previous
import functools

import jax
import jax.numpy as jnp
import numpy as np
from jax import lax
from jax.experimental import pallas as pl
from jax.experimental.pallas import tpu as pltpu
from jax.experimental.pallas import tpu_sc as plsc

B = 1
S = 2048
HID = 2048
QLORA = 1536
H = 32
D = 128
ROPE = 64
HALF = ROPE // 2
TOPK = 1024

QB = 512

NC = 2
NS = 16
NW = NC * NS
RPW = S // NW
LANES = 16
NCHUNK = S // LANES
NBINS = 256

_C = float(D) ** -0.5


def _hadamard(x, scale):
    orig_dtype = x.dtype
    x = x.astype(jnp.float32)
    dim = x.shape[-1]
    h = 1
    while h < dim:
        x = x.reshape(x.shape[:-1] + (dim // (2 * h), 2, h))
        a = x[..., 0, :]
        b = x[..., 1, :]
        x = jnp.stack([a + b, a - b], axis=-2)
        x = x.reshape(x.shape[:-3] + (dim,))
        h *= 2
    return (x * scale).astype(orig_dtype)


def _rotated_qk(hidden_states, q_compressed, cos, sin, Wq, Wk, ln_g, ln_b, Ww):
    b, s, _ = hidden_states.shape
    q = q_compressed @ Wq
    q = q.reshape(b, s, H, D)
    q_rope, q_nope = q[..., :ROPE], q[..., ROPE:]
    k = hidden_states @ Wk
    mu = jnp.mean(k, axis=-1, keepdims=True)
    var = jnp.mean((k - mu) ** 2, axis=-1, keepdims=True)
    k = (k - mu) / jnp.sqrt(var + 1e-5) * ln_g + ln_b
    k_rope, k_nope = k[..., :ROPE], k[..., ROPE:]
    k_rope = k_rope[:, :, None, :]
    cosu = cos[:, None, :]
    sinu = sin[:, None, :]
    q1, q2 = jnp.split(q_rope, 2, axis=-1)
    k1, k2 = jnp.split(k_rope, 2, axis=-1)
    q_rope = jnp.concatenate([q1 * cosu - q2 * sinu, q1 * sinu + q2 * cosu],
                             axis=-1)
    k_rope = jnp.concatenate([k1 * cosu - k2 * sinu, k1 * sinu + k2 * cosu],
                             axis=-1)
    k_rope = k_rope[:, :, 0, :]
    q = jnp.concatenate([q_rope, q_nope], axis=-1)
    k = jnp.concatenate([k_rope, k_nope], axis=-1)
    q = _hadamard(q, _C)
    k = _hadamard(k, _C)
    w = (hidden_states.astype(jnp.float32) @ Ww) * (H ** -0.5)
    qb = q.astype(jnp.bfloat16).reshape(s, H * D)
    kb = k.astype(jnp.bfloat16).reshape(s, D)
    return qb, kb, w.reshape(s, H)


def _scores_body(qb_ref, kb_ref, w_ref, out_ref):
    kb = kb_ref[...]
    w = w_ref[...]
    acc = None
    for h in range(H):
        qh = qb_ref[:, h * D:(h + 1) * D]
        sc = lax.dot_general(qh, kb, (((1,), (1,)), ((), ())),
                             preferred_element_type=jnp.float32)
        term = (jnp.maximum(sc, 0.0) * _C) * w[:, h:h + 1]
        acc = term if acc is None else acc + term
    bits = lax.bitcast_convert_type(acc, jnp.int32)
    bits = jnp.where(acc == 0.0, 0, bits)
    out_ref[...] = jnp.where(acc >= 0.0, jnp.int32(0x7FFFFFFF) - bits, bits)


def _keys_part(qb_part, kb, w_part, nrows):
    return pl.pallas_call(
        _scores_body,
        grid=(nrows // QB,),
        in_specs=[
            pl.BlockSpec((QB, H * D), lambda i: (i, 0)),
            pl.BlockSpec((S, D), lambda i: (0, 0)),
            pl.BlockSpec((QB, H), lambda i: (i, 0)),
        ],
        out_specs=pl.BlockSpec((QB, S), lambda i: (i, 0)),
        out_shape=jax.ShapeDtypeStruct((nrows, S), jnp.int32),
    )(qb_part, kb, w_part)


def _sc_topk_body(keys_hbm, out_hbm, key_a, val_a, key_b, val_b, cnt, tot,
                  sem):
    wid = lax.axis_index("s") * NC + lax.axis_index("c")
    lane = lax.iota(jnp.int32, LANES)
    zeros16 = jnp.zeros((LANES,), jnp.int32)
    ones16 = jnp.ones((LANES,), jnp.int32)

    def histogram(src, shift):
        shift_v = jnp.full((LANES,), shift, jnp.int32)

        @pl.loop(0, NBINS, unroll=8)
        def _(j):
            cnt[pl.ds(j * LANES, LANES)] = zeros16

        @pl.loop(0, NCHUNK, unroll=8)
        def _(ci):
            k = src[pl.ds(ci * LANES, LANES)]
            d = lax.shift_right_logical(k, shift_v) & (NBINS - 1)
            plsc.addupdate_scatter(cnt, [d * LANES + lane], ones16)

    def scan():
        @pl.loop(0, NBINS, unroll=8)
        def _(j):
            tot[j] = jnp.sum(cnt[pl.ds(j * LANES, LANES)])

        def tscan(j, carry):
            t = tot[j]
            tot[j] = carry
            return carry + t

        lax.fori_loop(0, NBINS, tscan, jnp.int32(0), unroll=8)

        @pl.loop(0, NBINS, unroll=8)
        def _(j):
            v = cnt[pl.ds(j * LANES, LANES)]
            cnt[pl.ds(j * LANES, LANES)] = plsc.cumsum(v) - v + tot[j]

    def fetch_slot(d):
        addr = d * LANES + lane
        slot = plsc.load_gather(cnt, [addr])
        plsc.store_scatter(cnt, [addr], slot + 1)
        return slot

    def sigma(slot):
        return (slot & (NCHUNK - 1)) * LANES + (slot >> 7)

    rpw = keys_hbm.shape[0] // NW

    @pl.loop(0, rpw)
    def _(ri):
        r = wid * rpw + ri
        pltpu.sync_copy(keys_hbm.at[r], key_a)

        histogram(key_a, 0)
        scan()

        @pl.loop(0, NCHUNK, unroll=8)
        def _(ci):
            k = key_a[pl.ds(ci * LANES, LANES)]
            pos = sigma(fetch_slot(k & (NBINS - 1)))
            plsc.store_scatter(key_b, [pos], k)
            plsc.store_scatter(val_b, [pos], lane + ci * LANES)

        histogram(key_b, 8)
        scan()
        shift8 = jnp.full((LANES,), 8, jnp.int32)

        @pl.loop(0, NCHUNK, unroll=8)
        def _(ci):
            k = key_b[pl.ds(ci * LANES, LANES)]
            v = val_b[pl.ds(ci * LANES, LANES)]
            d = lax.shift_right_logical(k, shift8) & (NBINS - 1)
            pos = sigma(fetch_slot(d))
            packed = lax.shift_left(
                lax.shift_right_logical(k, jnp.full((LANES,), 16, jnp.int32)),
                jnp.full((LANES,), 11, jnp.int32)) | v
            plsc.store_scatter(key_a, [pos], packed)

        histogram(key_a, 11)
        scan()
        shift11 = jnp.full((LANES,), 11, jnp.int32)

        @pl.loop(0, NCHUNK, unroll=8)
        def _(ci):
            p = key_a[pl.ds(ci * LANES, LANES)]
            d = lax.shift_right_logical(p, shift11) & (NBINS - 1)
            pos = sigma(fetch_slot(d))
            plsc.store_scatter(key_b, [pos], p)

        histogram(key_b, 19)
        scan()
        shift19 = jnp.full((LANES,), 19, jnp.int32)

        @pl.loop(0, NCHUNK, unroll=8)
        def _(ci):
            p = key_b[pl.ds(ci * LANES, LANES)]
            d = lax.shift_right_logical(p, shift19) & (NBINS - 1)
            slot = fetch_slot(d)
            plsc.store_scatter(val_a, [slot], p & 0x7FF)

        pltpu.sync_copy(val_a.at[pl.ds(0, TOPK)], out_hbm.at[r])


def _sc_topk(keys):
    mesh = plsc.VectorSubcoreMesh(core_axis_name="c", subcore_axis_name="s",
                                  num_cores=NC, num_subcores=NS)
    f = pl.kernel(
        _sc_topk_body,
        out_type=jax.ShapeDtypeStruct((keys.shape[0], TOPK), jnp.int32),
        mesh=mesh,
        compiler_params=pltpu.CompilerParams(needs_layout_passes=False),
        scratch_types=[
            pltpu.VMEM((S,), jnp.int32),
            pltpu.VMEM((S,), jnp.int32),
            pltpu.VMEM((S,), jnp.int32),
            pltpu.VMEM((S,), jnp.int32),
            pltpu.VMEM((NBINS * LANES,), jnp.int32),
            pltpu.SMEM((NBINS,), jnp.int32),
            pltpu.SemaphoreType.DMA,
        ],
    )
    return f(keys)


def kernel(hidden_states, q_compressed, cos, sin, Wq, Wk, ln_g, ln_b, Ww):
    qb, kb, w = _rotated_qk(hidden_states, q_compressed, cos, sin, Wq, Wk,
                            ln_g, ln_b, Ww)
    half = S // 2
    parts = []
    for p in range(2):
        sl = slice(p * half, (p + 1) * half)
        keys_p = _keys_part(qb[sl], kb, w[sl], half)
        parts.append(_sc_topk(keys_p))
    idx = jnp.concatenate(parts, axis=0)
    return idx.reshape(B, S, TOPK)
